# baseline (speedup 1.0000x reference)
import functools
import math

import jax
import jax.numpy as jnp
import numpy as np
from jax import lax
from jax.experimental import pallas as pl
from jax.experimental.pallas import tpu as pltpu
from jax.experimental.pallas import tpu_sc as plsc

_HIDDEN = 768
_NC, _NS = 2, 16
_NW = _NC * _NS
_K = 2
_CH = 64
_BR = 1024


def _pe_row(seq_len: int, hidden: int) -> np.ndarray:
    norm = np.exp(np.arange(0, hidden, 2, dtype=np.float64)
                  * (-(math.log(10000.0) / hidden)))
    row = np.zeros((hidden,), dtype=np.float64)
    row[0::2] = np.sin(seq_len * norm)
    row[1::2] = np.cos(seq_len * norm)
    return row.astype(np.float32)


def _sc_gather_chunk(table, ids, c, n_all):
    l = ids.shape[1]
    n = n_all // _K
    b_per_w = n // _NW
    n_ch = b_per_w // _CH
    mesh = plsc.VectorSubcoreMesh(core_axis_name="c", subcore_axis_name="s")

    @functools.partial(
        pl.kernel,
        mesh=mesh,
        out_type=jax.ShapeDtypeStruct((n, _HIDDEN), jnp.float32),
        scratch_types=[
            pltpu.VMEM((b_per_w,), jnp.int32),
            pltpu.VMEM((_CH, _HIDDEN), jnp.float32),
            pltpu.VMEM((_CH, _HIDDEN), jnp.float32),
            pltpu.SemaphoreType.DMA,
            pltpu.SemaphoreType.DMA,
            pltpu.SemaphoreType.DMA,
            pltpu.SemaphoreType.DMA,
        ],
    )
    def k(table_hbm, ids_hbm, out_hbm, idx_v, buf0, buf1,
          gsem0, gsem1, wsem0, wsem1):
        wid = lax.axis_index("s") * _NC + lax.axis_index("c")
        base = wid * b_per_w
        flat = c * n + base
        pltpu.sync_copy(ids_hbm.at[flat // l, pl.ds(flat % l, b_per_w)], idx_v)
        bufs = (buf0, buf1)
        gsems = (gsem0, gsem1)
        wsems = (wsem0, wsem1)
        gcp = [pltpu.async_copy(
            table_hbm.at[idx_v.at[pl.ds(i * _CH, _CH)]], bufs[i], gsems[i])
            for i in range(min(2, n_ch))]
        wcp = []
        for i in range(n_ch):
            gcp[i].wait()
            wcp.append(pltpu.async_copy(
                bufs[i % 2], out_hbm.at[pl.ds(base + i * _CH, _CH)],
                wsems[i % 2]))
            if i + 2 < n_ch:
                wcp[i].wait()
                gcp.append(pltpu.async_copy(
                    table_hbm.at[idx_v.at[pl.ds((i + 2) * _CH, _CH)]],
                    bufs[i % 2], gsems[i % 2]))
        if n_ch >= 2:
            wcp[-2].wait()
        wcp[-1].wait()

    return k(table, ids)


def _tc_ln_chunk(rows, seq_col, bias0, dbias, gamma, beta, c, out_prev):
    nc = rows.shape[0]
    n = seq_col.shape[0]
    grid = (nc // _BR,)
    base = c * (nc // _BR)

    def body(rows_ref, seq_ref, b0_ref, db_ref, g_ref, be_ref, *rest):
        out_ref = rest[-1]
        x = rows_ref[...]
        s = seq_ref[...].astype(jnp.float32)
        x = x + b0_ref[...] + s * db_ref[...]
        mean = jnp.mean(x, axis=-1, keepdims=True)
        xc = x - mean
        var = jnp.mean(xc * xc, axis=-1, keepdims=True)
        rstd = lax.rsqrt(var + 1e-12)
        out_ref[...] = g_ref[...] * (xc * rstd) + be_ref[...]

    in_specs = [
        pl.BlockSpec((_BR, _HIDDEN), lambda i: (i, 0)),
        pl.BlockSpec((_BR, 1), lambda i: (base + i, 0)),
        pl.BlockSpec((1, _HIDDEN), lambda i: (0, 0)),
        pl.BlockSpec((1, _HIDDEN), lambda i: (0, 0)),
        pl.BlockSpec((1, _HIDDEN), lambda i: (0, 0)),
        pl.BlockSpec((1, _HIDDEN), lambda i: (0, 0)),
    ]
    args = [rows, seq_col, bias0, dbias, gamma, beta]
    aliases = {}
    if out_prev is not None:
        in_specs.append(pl.BlockSpec(memory_space=pl.ANY))
        args.append(out_prev)
        aliases = {6: 0}
    return pl.pallas_call(
        body,
        grid=grid,
        in_specs=in_specs,
        out_specs=pl.BlockSpec((_BR, _HIDDEN), lambda i: (base + i, 0)),
        out_shape=jax.ShapeDtypeStruct((n, _HIDDEN), jnp.float32),
        input_output_aliases=aliases,
    )(*args)


def kernel(inputIDs, sequenceIDs, word_table, seq_table, gamma, beta):
    b, l = inputIDs.shape
    n = b * l
    ids = inputIDs.astype(jnp.int32)
    seq_col = sequenceIDs.reshape(n, 1).astype(jnp.int32)

    pe = jnp.asarray(_pe_row(l, _HIDDEN))
    bias0 = (seq_table[0] + pe).reshape(1, _HIDDEN)
    dbias = (seq_table[1] - seq_table[0]).reshape(1, _HIDDEN)
    gamma2 = gamma.reshape(1, _HIDDEN)
    beta2 = beta.reshape(1, _HIDDEN)

    gathered = [_sc_gather_chunk(word_table, ids, c, n) for c in range(_K)]
    out = None
    for c in range(_K):
        out = _tc_ln_chunk(gathered[c], seq_col, bias0, dbias,
                           gamma2, beta2, c, out)
    return out.reshape(b, l, _HIDDEN)


# speedup vs baseline: 1.0165x; 1.0165x over previous
import functools
import math

import jax
import jax.numpy as jnp
import numpy as np
from jax import lax
from jax.experimental import pallas as pl
from jax.experimental.pallas import tpu as pltpu
from jax.experimental.pallas import tpu_sc as plsc

_HIDDEN = 768
_NC, _NS = 2, 16
_NW = _NC * _NS
_K = 1
_CH = 32
_NBUF = 4
_BR = 1024


def _pe_row(seq_len: int, hidden: int) -> np.ndarray:
    norm = np.exp(np.arange(0, hidden, 2, dtype=np.float64)
                  * (-(math.log(10000.0) / hidden)))
    row = np.zeros((hidden,), dtype=np.float64)
    row[0::2] = np.sin(seq_len * norm)
    row[1::2] = np.cos(seq_len * norm)
    return row.astype(np.float32)


def _sc_gather_chunk(table, ids, c, n_all):
    l = ids.shape[1]
    n = n_all // _K
    b_per_w = n // _NW
    n_ch = b_per_w // _CH
    mesh = plsc.VectorSubcoreMesh(core_axis_name="c", subcore_axis_name="s")

    @functools.partial(
        pl.kernel,
        mesh=mesh,
        out_type=jax.ShapeDtypeStruct((n, _HIDDEN), jnp.float32),
        scratch_types=(
            [pltpu.VMEM((b_per_w,), jnp.int32)]
            + [pltpu.VMEM((_CH, _HIDDEN), jnp.float32)] * _NBUF
            + [pltpu.SemaphoreType.DMA] * (2 * _NBUF)
        ),
    )
    def k(table_hbm, ids_hbm, out_hbm, idx_v, *rest):
        bufs = rest[:_NBUF]
        gsems = rest[_NBUF:2 * _NBUF]
        wsems = rest[2 * _NBUF:]
        wid = lax.axis_index("s") * _NC + lax.axis_index("c")
        base = wid * b_per_w
        flat = c * n + base
        pltpu.sync_copy(ids_hbm.at[flat // l, pl.ds(flat % l, b_per_w)], idx_v)

        def gath(i):
            return pltpu.async_copy(
                table_hbm.at[idx_v.at[pl.ds(i * _CH, _CH)]],
                bufs[i % _NBUF], gsems[i % _NBUF])

        gcp = [gath(i) for i in range(min(_NBUF, n_ch))]
        wcp = []
        for i in range(n_ch):
            gcp[i].wait()
            wcp.append(pltpu.async_copy(
                bufs[i % _NBUF], out_hbm.at[pl.ds(base + i * _CH, _CH)],
                wsems[i % _NBUF]))
            j = i - 1
            if j >= 0 and j + _NBUF < n_ch:
                wcp[j].wait()
                gcp.append(gath(j + _NBUF))
        for i in range(max(0, n_ch - _NBUF), n_ch):
            wcp[i].wait()

    return k(table, ids)


def _tc_ln_chunk(rows, seq_col, bias0, dbias, gamma, beta, c, out_prev):
    nc = rows.shape[0]
    n = seq_col.shape[0]
    grid = (nc // _BR,)
    base = c * (nc // _BR)

    def body(rows_ref, seq_ref, b0_ref, db_ref, g_ref, be_ref, *rest):
        out_ref = rest[-1]
        x = rows_ref[...]
        s = seq_ref[...].astype(jnp.float32)
        x = x + b0_ref[...] + s * db_ref[...]
        mean = jnp.mean(x, axis=-1, keepdims=True)
        xc = x - mean
        var = jnp.mean(xc * xc, axis=-1, keepdims=True)
        rstd = lax.rsqrt(var + 1e-12)
        out_ref[...] = g_ref[...] * (xc * rstd) + be_ref[...]

    in_specs = [
        pl.BlockSpec((_BR, _HIDDEN), lambda i: (i, 0)),
        pl.BlockSpec((_BR, 1), lambda i: (base + i, 0)),
        pl.BlockSpec((1, _HIDDEN), lambda i: (0, 0)),
        pl.BlockSpec((1, _HIDDEN), lambda i: (0, 0)),
        pl.BlockSpec((1, _HIDDEN), lambda i: (0, 0)),
        pl.BlockSpec((1, _HIDDEN), lambda i: (0, 0)),
    ]
    args = [rows, seq_col, bias0, dbias, gamma, beta]
    aliases = {}
    if out_prev is not None:
        in_specs.append(pl.BlockSpec(memory_space=pl.ANY))
        args.append(out_prev)
        aliases = {6: 0}
    return pl.pallas_call(
        body,
        grid=grid,
        in_specs=in_specs,
        out_specs=pl.BlockSpec((_BR, _HIDDEN), lambda i: (base + i, 0)),
        out_shape=jax.ShapeDtypeStruct((n, _HIDDEN), jnp.float32),
        input_output_aliases=aliases,
    )(*args)


def kernel(inputIDs, sequenceIDs, word_table, seq_table, gamma, beta):
    b, l = inputIDs.shape
    n = b * l
    ids = inputIDs.astype(jnp.int32)
    seq_col = sequenceIDs.reshape(n, 1).astype(jnp.int32)

    pe = jnp.asarray(_pe_row(l, _HIDDEN))
    bias0 = (seq_table[0] + pe).reshape(1, _HIDDEN)
    dbias = (seq_table[1] - seq_table[0]).reshape(1, _HIDDEN)
    gamma2 = gamma.reshape(1, _HIDDEN)
    beta2 = beta.reshape(1, _HIDDEN)

    gathered = [_sc_gather_chunk(word_table, ids, c, n) for c in range(_K)]
    out = None
    for c in range(_K):
        out = _tc_ln_chunk(gathered[c], seq_col, bias0, dbias,
                           gamma2, beta2, c, out)
    return out.reshape(b, l, _HIDDEN)
